# baseline (speedup 1.0000x reference)
import functools
import math

import jax
import jax.numpy as jnp
from jax import lax
from jax.experimental import pallas as pl
from jax.experimental.pallas import tpu as pltpu
from jax.experimental.pallas import tpu_sc as plsc

VOCAB = 1000000
DIM = 64
BATCH = 4096
SEQ = 200
SCALE = math.sqrt(DIM)

NC = 2
NS = 16
NW = NC * NS

B_TOTAL = BATCH * SEQ
B_PER_W = B_TOTAL // NW
CHUNK = 512
N_CHUNKS = B_PER_W // CHUNK
IDX_SLICE = 128
N_GATHERS = CHUNK // IDX_SLICE
LANES = 16


@functools.partial(
    pl.kernel,
    out_type=jax.ShapeDtypeStruct((B_TOTAL, DIM), jnp.float32),
    mesh=plsc.VectorSubcoreMesh(core_axis_name="c", subcore_axis_name="s"),
    scratch_types=[
        pltpu.VMEM((B_PER_W,), jnp.int32),
        pltpu.VMEM((CHUNK, DIM), jnp.float32),
        pltpu.SemaphoreType.DMA,
    ],
    compiler_params=pltpu.CompilerParams(use_tc_tiling_on_sc=False),
)
def _emb_lookup(x_hbm, table_hbm, out_hbm, idx_v, rows_v, gsem):
    wid = lax.axis_index("s") * NC + lax.axis_index("c")
    base = wid * B_PER_W

    pltpu.sync_copy(x_hbm.at[pl.ds(base, B_PER_W)], idx_v)

    def chunk_body(g, carry):
        copies = []
        for j in range(N_GATHERS):
            copies.append(
                pltpu.async_copy(
                    table_hbm.at[idx_v.at[pl.ds(g * CHUNK + j * IDX_SLICE, IDX_SLICE)]],
                    rows_v.at[pl.ds(j * IDX_SLICE, IDX_SLICE)],
                    gsem,
                )
            )
        for c in copies:
            c.wait()

        def row_body(r, c2):
            for k in range(DIM // LANES):
                v = rows_v[r, pl.ds(k * LANES, LANES)]
                rows_v[r, pl.ds(k * LANES, LANES)] = v * SCALE
            return c2

        lax.fori_loop(0, CHUNK, row_body, 0)

        pltpu.sync_copy(rows_v, out_hbm.at[pl.ds(base + g * CHUNK, CHUNK)])
        return carry

    lax.fori_loop(0, N_CHUNKS, chunk_body, 0)


def kernel(x, table):
    xf = x.reshape(-1).astype(jnp.int32)
    out = _emb_lookup(xf, table)
    return out.reshape(x.shape[0], x.shape[1], DIM)


# speedup vs baseline: 1.1160x; 1.1160x over previous
import functools
import math

import jax
import jax.numpy as jnp
from jax import lax
from jax.experimental import pallas as pl
from jax.experimental.pallas import tpu as pltpu
from jax.experimental.pallas import tpu_sc as plsc

VOCAB = 1000000
DIM = 64
BATCH = 4096
SEQ = 200
SCALE = math.sqrt(DIM)

NC = 2
NS = 16
NW = NC * NS

B_TOTAL = BATCH * SEQ
B_PER_W = B_TOTAL // NW
CHUNK = 256
N_CHUNKS = B_PER_W // CHUNK
NBUF = 4
LANES = 16
ROW_UNROLL = 4


@functools.partial(
    pl.kernel,
    out_type=jax.ShapeDtypeStruct((B_TOTAL, DIM), jnp.float32),
    mesh=plsc.VectorSubcoreMesh(core_axis_name="c", subcore_axis_name="s"),
    scratch_types=[
        pltpu.VMEM((B_PER_W,), jnp.int32),
        [pltpu.VMEM((CHUNK, DIM), jnp.float32) for _ in range(NBUF)],
        [pltpu.SemaphoreType.DMA for _ in range(NBUF)],
        [pltpu.SemaphoreType.DMA for _ in range(NBUF)],
    ],
    compiler_params=pltpu.CompilerParams(use_tc_tiling_on_sc=False),
)
def _emb_lookup(x_hbm, table_hbm, out_hbm, idx_v, rows, gsems, ssems):
    wid = lax.axis_index("s") * NC + lax.axis_index("c")
    base = wid * B_PER_W

    pltpu.sync_copy(x_hbm.at[pl.ds(base, B_PER_W)], idx_v)

    def start_gather(g, slot):
        pltpu.async_copy(
            table_hbm.at[idx_v.at[pl.ds(g * CHUNK, CHUNK)]],
            rows[slot],
            gsems[slot],
        )

    def wait_gather(g, slot):
        pltpu.make_async_copy(
            table_hbm.at[idx_v.at[pl.ds(g * CHUNK, CHUNK)]],
            rows[slot],
            gsems[slot],
        ).wait()

    def start_scatter(g, slot):
        pltpu.async_copy(
            rows[slot],
            out_hbm.at[pl.ds(base + g * CHUNK, CHUNK)],
            ssems[slot],
        )

    def wait_scatter(g, slot):
        pltpu.make_async_copy(
            rows[slot],
            out_hbm.at[pl.ds(base + g * CHUNK, CHUNK)],
            ssems[slot],
        ).wait()

    def scale_rows(slot):
        def row_body(r, c):
            for rr in range(ROW_UNROLL):
                for k in range(DIM // LANES):
                    sl = pl.ds(k * LANES, LANES)
                    row = r * ROW_UNROLL + rr
                    rows[slot][row, sl] = rows[slot][row, sl] * SCALE
            return c

        lax.fori_loop(0, CHUNK // ROW_UNROLL, row_body, 0)

    start_gather(0, 0)
    start_gather(1, 1)

    def outer(i, carry):
        g0 = i * NBUF
        for b in range(NBUF):
            g = g0 + b
            nslot = (b + 2) % NBUF
            gn = g + 2

            @pl.when(jnp.logical_and(gn < N_CHUNKS, g >= 2))
            def _():
                wait_scatter(g - 2, nslot)

            @pl.when(gn < N_CHUNKS)
            def _():
                start_gather(gn, nslot)

            wait_gather(g, b)
            scale_rows(b)
            start_scatter(g, b)
        return carry

    lax.fori_loop(0, N_CHUNKS // NBUF, outer, 0)

    for b in range(NBUF):
        wait_scatter(N_CHUNKS - NBUF + b, b)


def kernel(x, table):
    xf = x.reshape(-1).astype(jnp.int32)
    out = _emb_lookup(xf, table)
    return out.reshape(x.shape[0], x.shape[1], DIM)
